# baseline (speedup 1.0000x reference)
import functools

import jax
import jax.numpy as jnp
from jax import lax
from jax.experimental import pallas as pl
from jax.experimental.pallas import tpu as pltpu
from jax.experimental.pallas import tpu_sc as plsc

NC = 2
NS = 16
NW = NC * NS

BLK = 128
NBUF = 8


def _make_gather(H, N, V, D):
    B = H * N
    n_blocks = B // BLK
    bpw = n_blocks // NW
    DT = D // 8
    assert n_blocks % NW == 0 and bpw > NBUF

    mesh = plsc.VectorSubcoreMesh(
        core_axis_name="c", subcore_axis_name="s",
        num_cores=NC, num_subcores=NS)

    @functools.partial(
        pl.kernel,
        out_type=(jax.ShapeDtypeStruct((H, DT, N // BLK, 8, BLK), jnp.float32),
                  jax.ShapeDtypeStruct((1024,), jnp.float32)),
        mesh=mesh,
        scratch_types=[
            pltpu.VMEM((bpw * BLK,), jnp.int32),
            pltpu.VMEM((NBUF * BLK, D), jnp.float32),
            pltpu.VMEM((NBUF * D, BLK + 1), jnp.float32),
            pltpu.SemaphoreType.DMA((NBUF,)),
            pltpu.SemaphoreType.DMA((NBUF,)),
            pltpu.VMEM((1024,), jnp.float32),
        ],
        compiler_params=pltpu.CompilerParams(
            use_tc_tiling_on_sc=False, needs_layout_passes=False),
    )
    def gather_kernel(idx_hbm, table_hbm, tblT_hbm, out_hbm, probe_hbm, idx_v, buf, bufT, gsem, ssem, pv):
        pltpu.sync_copy(tblT_hbm.at[0, pl.ds(0, 1024)], pv)
        pltpu.sync_copy(pv, probe_hbm.at[pl.ds(0, 1024)])
        wid = lax.axis_index("s") * NC + lax.axis_index("c")
        blk_base = wid * bpw

        pltpu.sync_copy(idx_hbm.at[pl.ds(blk_base * BLK, bpw * BLK)], idx_v)

        def fire_gather(j, b):
            pltpu.async_copy(
                table_hbm.at[idx_v.at[pl.ds(j * BLK, BLK)]],
                buf.at[pl.ds(b * BLK, BLK), :],
                gsem.at[b])

        def drain_gather(b):
            pltpu.make_async_copy(
                table_hbm.at[idx_v.at[pl.ds(0, BLK)]],
                buf.at[pl.ds(b * BLK, BLK), :],
                gsem.at[b]).wait()

        def fire_stores(j, b):
            g = blk_base + j
            h = g // (N // BLK)
            tc = g % (N // BLK)
            for tr in range(DT):
                pltpu.async_copy(
                    bufT.at[pl.ds(b * D + tr * 8, 8), pl.ds(0, BLK)],
                    out_hbm.at[h, tr, tc],
                    ssem.at[b])

        def wait_stores(b):
            for tr in range(DT):
                pltpu.make_async_copy(
                    bufT.at[pl.ds(tr * 8, 8), pl.ds(0, BLK)],
                    out_hbm.at[0, tr, 0],
                    ssem.at[b]).wait()

        iota16 = lax.iota(jnp.int32, 16)
        ones16 = jnp.full((16,), 1, jnp.int32)

        def transpose_block(b):
            d_lo = iota16 + b * D
            d_hi = d_lo + 16
            nv = jnp.full((16,), 0, jnp.int32)
            for n in range(BLK):
                row = b * BLK + n
                v0 = buf[row, pl.ds(0, 16)]
                v1 = buf[row, pl.ds(16, 16)]
                plsc.store_scatter(bufT, [d_lo, nv], v0)
                plsc.store_scatter(bufT, [d_hi, nv], v1)
                if n + 1 < BLK:
                    nv = nv + ones16

        for j in range(NBUF):
            fire_gather(j, j)

        def body(j, carry):
            b = j % NBUF
            drain_gather(b)
            jp = j - 1 + NBUF

            @pl.when(jnp.logical_and(j >= 1, jp < bpw))
            def _():
                bp = (j - 1) % NBUF
                wait_stores(bp)
                fire_gather(jp, bp)

            transpose_block(b)
            fire_stores(j, b)
            return carry

        lax.fori_loop(0, bpw, body, 0)

        for j in range(bpw - NBUF, bpw):
            wait_stores(j % NBUF)

    return gather_kernel


def kernel(inputs, table):
    n, h = inputs.shape
    V, D = table.shape
    idx = inputs.T.reshape(n * h).astype(jnp.int32)
    tbl = table.reshape(V * D // 128, 128)
    tbl = lax.optimization_barrier(tbl)
    tbl = tbl.reshape(V, D)
    out5, _probe = _make_gather(h, n, V, D)(idx, tbl, table.T)
    return out5.transpose(2, 4, 0, 1, 3).reshape(n, h, D)


# speedup vs baseline: 4.6493x; 4.6493x over previous
import functools

import jax
import jax.numpy as jnp
from jax import lax
from jax.experimental import pallas as pl
from jax.experimental.pallas import tpu as pltpu
from jax.experimental.pallas import tpu_sc as plsc

NC = 2
NS = 16
NW = NC * NS

BLK = 128
NBUF = 8


def _make_gather(H, N, V, D):
    B = H * N
    n_blocks = B // BLK
    bpw = n_blocks // NW
    DT = D // 8
    assert n_blocks % NW == 0 and bpw > NBUF

    mesh = plsc.VectorSubcoreMesh(
        core_axis_name="c", subcore_axis_name="s",
        num_cores=NC, num_subcores=NS)

    @functools.partial(
        pl.kernel,
        out_type=jax.ShapeDtypeStruct((H, DT, N // BLK, 8, BLK), jnp.float32),
        mesh=mesh,
        scratch_types=[
            pltpu.VMEM((bpw * BLK,), jnp.int32),
            pltpu.VMEM((NBUF * BLK, D), jnp.float32),
            pltpu.VMEM((NBUF * D, BLK + 1), jnp.float32),
            pltpu.SemaphoreType.DMA((NBUF,)),
            pltpu.SemaphoreType.DMA((NBUF,)),
        ],
        compiler_params=pltpu.CompilerParams(
            use_tc_tiling_on_sc=False, needs_layout_passes=False),
    )
    def gather_kernel(idx_hbm, table_hbm, out_hbm, idx_v, buf, bufT, gsem, ssem):
        wid = lax.axis_index("s") * NC + lax.axis_index("c")
        blk_base = wid * bpw

        pltpu.sync_copy(idx_hbm.at[pl.ds(blk_base * BLK, bpw * BLK)], idx_v)

        def fire_gather(j, b):
            pltpu.async_copy(
                table_hbm.at[idx_v.at[pl.ds(j * BLK, BLK)]],
                buf.at[pl.ds(b * BLK, BLK), :],
                gsem.at[b])

        def drain_gather(b):
            pltpu.make_async_copy(
                table_hbm.at[idx_v.at[pl.ds(0, BLK)]],
                buf.at[pl.ds(b * BLK, BLK), :],
                gsem.at[b]).wait()

        def fire_stores(j, b):
            g = blk_base + j
            h = g // (N // BLK)
            tc = g % (N // BLK)
            for tr in range(DT):
                pltpu.async_copy(
                    bufT.at[pl.ds(b * D + tr * 8, 8), pl.ds(0, BLK)],
                    out_hbm.at[h, tr, tc],
                    ssem.at[b])

        def wait_stores(b):
            for tr in range(DT):
                pltpu.make_async_copy(
                    bufT.at[pl.ds(tr * 8, 8), pl.ds(0, BLK)],
                    out_hbm.at[0, tr, 0],
                    ssem.at[b]).wait()

        iota16 = lax.iota(jnp.int32, 16)
        ones16 = jnp.full((16,), 1, jnp.int32)

        def transpose_block(b):
            d_lo = iota16 + b * D
            d_hi = d_lo + 16
            nv = jnp.full((16,), 0, jnp.int32)
            for n in range(BLK):
                row = b * BLK + n
                v0 = buf[row, pl.ds(0, 16)]
                v1 = buf[row, pl.ds(16, 16)]
                plsc.store_scatter(bufT, [d_lo, nv], v0)
                plsc.store_scatter(bufT, [d_hi, nv], v1)
                if n + 1 < BLK:
                    nv = nv + ones16

        for j in range(NBUF):
            fire_gather(j, j)

        def body(j, carry):
            b = j % NBUF
            drain_gather(b)
            jp = j - 1 + NBUF

            @pl.when(jnp.logical_and(j >= 1, jp < bpw))
            def _():
                bp = (j - 1) % NBUF
                wait_stores(bp)
                fire_gather(jp, bp)

            transpose_block(b)
            fire_stores(j, b)
            return carry

        lax.fori_loop(0, bpw, body, 0)

        for j in range(bpw - NBUF, bpw):
            wait_stores(j % NBUF)

    return gather_kernel


def kernel(inputs, table):
    n, h = inputs.shape
    V, D = table.shape
    idx = inputs.T.reshape(n * h).astype(jnp.int32)
    tbl = table.reshape(V * D // 128, 128)
    tbl = lax.optimization_barrier(tbl)
    tbl = tbl.reshape(V, D)
    out5 = _make_gather(h, n, V, D)(idx, tbl)
    return out5.transpose(2, 4, 0, 1, 3).reshape(n, h, D)
